# baseline (speedup 1.0000x reference)
import functools

import jax
import jax.numpy as jnp
from jax import lax
from jax.experimental import pallas as pl
from jax.experimental.pallas import tpu as pltpu
from jax.experimental.pallas import tpu_sc as plsc

_N_ATOMS = 100000
_EMB = 128
_INFO = plsc.get_sparse_core_info()
_NW = _INFO.num_cores * _INFO.num_subcores
_CHUNK = 128
_ROWS_PER_W = 3200
_N_PAD = _NW * _ROWS_PER_W
_N_CHUNKS = _ROWS_PER_W // _CHUNK


def _emb_body(z_hbm, w_hbm, out_hbm, idx_v, rows_v, sem):
    wid = lax.axis_index("s") * _INFO.num_cores + lax.axis_index("c")
    base = wid * _ROWS_PER_W

    def chunk(j, carry):
        row0 = base + j * _CHUNK
        pltpu.sync_copy(z_hbm.at[pl.ds(row0, _CHUNK)], idx_v)
        for i in range(_CHUNK // 16):
            sl = pl.ds(i * 16, 16)
            idx_v[sl] = idx_v[sl] - 1
        pltpu.async_copy(w_hbm.at[idx_v], rows_v, sem).wait()
        pltpu.sync_copy(rows_v, out_hbm.at[pl.ds(row0, _CHUNK)])
        return carry

    lax.fori_loop(0, _N_CHUNKS, chunk, 0)


@jax.jit
def kernel(Z, W):
    z_pad = jnp.concatenate([Z, jnp.ones((_N_PAD - _N_ATOMS,), jnp.int32)])
    mesh = plsc.VectorSubcoreMesh(core_axis_name="c", subcore_axis_name="s")
    out = pl.kernel(
        _emb_body,
        out_type=jax.ShapeDtypeStruct((_N_PAD, _EMB), jnp.float32),
        mesh=mesh,
        scratch_types=[
            pltpu.VMEM((_CHUNK,), jnp.int32),
            pltpu.VMEM((_CHUNK, _EMB), jnp.float32),
            pltpu.SemaphoreType.DMA,
        ],
    )(z_pad, W)
    return out[:_N_ATOMS]
